# Initial kernel scaffold; baseline (speedup 1.0000x reference)
import jax
import jax.numpy as jnp
from jax.experimental import pallas as pl


def kernel(X, norm1_w, norm2_w, W_q, W_k, W_v, W_o, router_W, expert_bias, sh_wg, sh_wu, sh_wd, ex_wg, ex_wu, ex_wd):
    raise NotImplementedError("write your pallas kernel here")


# speedup vs baseline: 1.7314x; 1.7314x over previous
import functools

import numpy as np
import jax
import jax.numpy as jnp
from jax import lax
from jax.experimental import pallas as pl
from jax.experimental.pallas import tpu as pltpu


def _bf(x):
    return x.astype(jnp.bfloat16)


def _f32dot(a, b):
    return jnp.dot(a, b, preferred_element_type=jnp.float32,
                   precision=jax.lax.Precision.HIGHEST)


def _bfdot(a, b):
    return jnp.dot(_bf(a), _bf(b), preferred_element_type=jnp.float32)


def _prep_kernel(x_ref, n1_ref, wq_ref, wk_ref, wv_ref, q_ref, k_ref, v_ref):
    x = x_ref[...]
    nx = x * lax.rsqrt(jnp.mean(x * x, axis=-1, keepdims=True) + 1e-6)
    nx = nx * n1_ref[...]
    nxb = _bf(nx)
    q_ref[...] = jnp.dot(nxb, _bf(wq_ref[...]), preferred_element_type=jnp.float32)
    k_ref[...] = jnp.dot(nxb, _bf(wk_ref[...]), preferred_element_type=jnp.float32)
    v_ref[...] = jnp.dot(nxb, _bf(wv_ref[...]), preferred_element_type=jnp.float32)


def _attn_kernel(q_ref, k_ref, v_ref, x_ref, wo_ref, o_ref, *, dk, tb):
    h = pl.program_id(0)
    t = pl.program_id(1)
    q = _bf(q_ref[...])
    kc = _bf(k_ref[...])
    s = lax.dot_general(q, kc, (((1,), (1,)), ((), ())),
                        preferred_element_type=jnp.float32)
    s = s * (1.0 / np.sqrt(dk))
    m = jnp.max(s, axis=-1, keepdims=True)
    p = jnp.exp(s - m)
    l = jnp.sum(p, axis=-1, keepdims=True)
    o = jnp.dot(_bf(p), _bf(v_ref[...]), preferred_element_type=jnp.float32) / l
    contrib = jnp.dot(_bf(o), _bf(wo_ref[...]), preferred_element_type=jnp.float32)
    rows = pl.ds(t * tb, tb)

    @pl.when(h == 0)
    def _():
        o_ref[rows, :] = x_ref[...] + contrib

    @pl.when(h > 0)
    def _():
        o_ref[rows, :] += contrib


def _router_kernel(x1_ref, n2_ref, rw_ref, bias_ref, nx2_ref, tw_ref, *, ne):
    x = x1_ref[...]
    nx = x * lax.rsqrt(jnp.mean(x * x, axis=-1, keepdims=True) + 1e-6)
    nx = nx * n2_ref[...]
    nx2_ref[...] = nx
    logits = _f32dot(nx, rw_ref[...]) + bias_ref[...]
    lm = jnp.max(logits, axis=-1, keepdims=True)
    el = jnp.exp(logits - lm)
    rw = el / jnp.sum(el, axis=-1, keepdims=True)
    t = rw.shape[0]
    iota = lax.broadcasted_iota(jnp.int32, (t, ne), 1)
    m1 = jnp.max(rw, axis=-1, keepdims=True)
    i1 = jnp.min(jnp.where(rw == m1, iota, ne), axis=-1, keepdims=True)
    mask1 = iota == i1
    rw2 = jnp.where(mask1, -jnp.inf, rw)
    m2 = jnp.max(rw2, axis=-1, keepdims=True)
    i2 = jnp.min(jnp.where(rw2 == m2, iota, ne), axis=-1, keepdims=True)
    mask2 = iota == i2
    e2 = jnp.exp(m2 - m1)
    w1 = 1.0 / (1.0 + e2)
    w2 = e2 / (1.0 + e2)
    tw_ref[...] = jnp.where(mask1, w1, 0.0) + jnp.where(mask2, w2, 0.0)


def _shared_kernel(x1_ref, nx2_ref, wg_ref, wu_ref, wd_ref, o_ref):
    x = _bf(nx2_ref[...])
    g = jnp.dot(x, _bf(wg_ref[...]), preferred_element_type=jnp.float32)
    u = jnp.dot(x, _bf(wu_ref[...]), preferred_element_type=jnp.float32)
    hdn = jax.nn.silu(g) * u
    o_ref[...] = x1_ref[...] + jnp.dot(_bf(hdn), _bf(wd_ref[...]),
                                       preferred_element_type=jnp.float32)


def _experts_dense_kernel(acc_ref, nx2_ref, tw_ref, wg_ref, wu_ref, wd_ref,
                          o_ref, *, ne, tb):
    e = pl.program_id(0)
    t = pl.program_id(1)
    x = _bf(nx2_ref[...])
    g = jnp.dot(x, _bf(wg_ref[0]), preferred_element_type=jnp.float32)
    u = jnp.dot(x, _bf(wu_ref[0]), preferred_element_type=jnp.float32)
    hdn = jax.nn.silu(g) * u
    contrib = jnp.dot(_bf(hdn), _bf(wd_ref[0]), preferred_element_type=jnp.float32)
    iota = lax.broadcasted_iota(jnp.int32, (tb, ne), 1)
    w = jnp.sum(jnp.where(iota == e, tw_ref[...], 0.0), axis=-1, keepdims=True)
    contrib = contrib * w
    rows = pl.ds(t * tb, tb)

    @pl.when(e == 0)
    def _():
        o_ref[rows, :] = acc_ref[...] + contrib

    @pl.when(e > 0)
    def _():
        o_ref[rows, :] += contrib


def kernel(X, norm1_w, norm2_w, W_q, W_k, W_v, W_o, router_W, expert_bias,
           sh_wg, sh_wu, sh_wd, ex_wg, ex_wu, ex_wd):
    b, s, d = X.shape
    ne, _, dff = ex_wg.shape
    h = 4
    dk = d // h
    dkv = d // 4
    f32 = jnp.float32

    Xf = X.reshape(b * s, d)
    n1 = norm1_w.reshape(1, d)
    n2 = norm2_w.reshape(1, d)
    bias = expert_bias.reshape(1, ne)
    T = b * s
    TB = 512
    nt = T // TB

    q, kc, vc = pl.pallas_call(
        _prep_kernel,
        out_shape=(jax.ShapeDtypeStruct((T, d), f32),
                   jax.ShapeDtypeStruct((T, dkv), f32),
                   jax.ShapeDtypeStruct((T, dkv), f32)),
    )(Xf, n1, W_q, W_k, W_v)

    x1 = pl.pallas_call(
        functools.partial(_attn_kernel, dk=dk, tb=TB),
        grid=(h, nt),
        in_specs=[
            pl.BlockSpec((TB, dk), lambda i, t: (t, i)),
            pl.BlockSpec((T, dkv), lambda i, t: (0, 0)),
            pl.BlockSpec((T, dkv), lambda i, t: (0, 0)),
            pl.BlockSpec((TB, d), lambda i, t: (t, 0)),
            pl.BlockSpec((dkv, d), lambda i, t: (i, 0)),
        ],
        out_specs=pl.BlockSpec((T, d), lambda i, t: (0, 0)),
        out_shape=jax.ShapeDtypeStruct((T, d), f32),
        compiler_params=pltpu.CompilerParams(
            dimension_semantics=("arbitrary", "arbitrary")),
    )(q, kc, vc, Xf, W_o)

    nx2, tw = pl.pallas_call(
        functools.partial(_router_kernel, ne=ne),
        out_shape=(jax.ShapeDtypeStruct((T, d), f32),
                   jax.ShapeDtypeStruct((T, ne), f32)),
    )(x1, n2, router_W, bias)

    acc = pl.pallas_call(
        _shared_kernel,
        grid=(nt,),
        in_specs=[
            pl.BlockSpec((TB, d), lambda t: (t, 0)),
            pl.BlockSpec((TB, d), lambda t: (t, 0)),
            pl.BlockSpec((d, dff), lambda t: (0, 0)),
            pl.BlockSpec((d, dff), lambda t: (0, 0)),
            pl.BlockSpec((dff, d), lambda t: (0, 0)),
        ],
        out_specs=pl.BlockSpec((TB, d), lambda t: (t, 0)),
        out_shape=jax.ShapeDtypeStruct((T, d), f32),
    )(x1, nx2, sh_wg, sh_wu, sh_wd)

    out = pl.pallas_call(
        functools.partial(_experts_dense_kernel, ne=ne, tb=TB),
        grid=(ne, nt),
        in_specs=[
            pl.BlockSpec((TB, d), lambda e, t: (t, 0)),
            pl.BlockSpec((TB, d), lambda e, t: (t, 0)),
            pl.BlockSpec((TB, ne), lambda e, t: (t, 0)),
            pl.BlockSpec((1, d, dff), lambda e, t: (e, 0, 0)),
            pl.BlockSpec((1, d, dff), lambda e, t: (e, 0, 0)),
            pl.BlockSpec((1, dff, d), lambda e, t: (e, 0, 0)),
        ],
        out_specs=pl.BlockSpec((T, d), lambda e, t: (0, 0)),
        out_shape=jax.ShapeDtypeStruct((T, d), f32),
        compiler_params=pltpu.CompilerParams(
            dimension_semantics=("arbitrary", "arbitrary")),
    )(acc, nx2, tw, ex_wg, ex_wu, ex_wd)

    return out.reshape(b, s, d)
